# baseline (speedup 1.0000x reference)
import functools

import jax
import jax.numpy as jnp
from jax import lax
from jax.experimental import pallas as pl
from jax.experimental.pallas import tpu as pltpu
from jax.experimental.pallas import tpu_sc as plsc

W_DIM = 64
BATCH = 4096
SEQ = 200
NUM_IDX = BATCH * SEQ
NC, NS = 2, 16
NW = NC * NS
CHUNK = 128
ROWS = NUM_IDX // CHUNK
RPW = ROWS // NW
K = 8
G = RPW // K


def _gather_body(idx_hbm, table_hbm, out_hbm, idx_v, rows_v, sem):
    wid = lax.axis_index("s") * NC + lax.axis_index("c")
    base_row = wid * RPW

    def group(g, carry):
        r0 = base_row + g * K
        pltpu.sync_copy(idx_hbm.at[pl.ds(r0, K)], idx_v)
        copies = []
        for j in range(K):
            copies.append(
                pltpu.async_copy(
                    table_hbm.at[idx_v.at[j]],
                    rows_v.at[pl.ds(j * CHUNK, CHUNK)],
                    sem,
                )
            )
        for c in copies:
            c.wait()
        pltpu.sync_copy(rows_v, out_hbm.at[pl.ds(r0 * CHUNK, K * CHUNK)])
        return carry

    lax.fori_loop(0, G, group, 0)


_sc_gather = functools.partial(
    pl.kernel,
    mesh=plsc.VectorSubcoreMesh(core_axis_name="c", subcore_axis_name="s"),
    out_type=jax.ShapeDtypeStruct((NUM_IDX, W_DIM), jnp.float32),
    scratch_types=[
        pltpu.VMEM((K, CHUNK), jnp.int32),
        pltpu.VMEM((K * CHUNK, W_DIM), jnp.float32),
        pltpu.SemaphoreType.DMA,
    ],
    compiler_params=pltpu.CompilerParams(use_tc_tiling_on_sc=False),
)(_gather_body)


def _mask_body(idx_ref, mask_ref):
    mask_ref[...] = idx_ref[...] != 0


def _make_mask(idx2d):
    return pl.pallas_call(
        _mask_body,
        out_shape=jax.ShapeDtypeStruct(idx2d.shape, jnp.bool_),
    )(idx2d)


def kernel(input_var, w_embedding):
    idx2d = input_var.reshape(ROWS, CHUNK)
    embedded = _sc_gather(idx2d, w_embedding)
    mask = _make_mask(idx2d)
    embedded = embedded.reshape(BATCH, SEQ, W_DIM)
    src_mask = mask.reshape(BATCH, SEQ)[:, None, None, :]
    return (embedded, src_mask)


# speedup vs baseline: 1.0147x; 1.0147x over previous
import functools

import jax
import jax.numpy as jnp
from jax import lax
from jax.experimental import pallas as pl
from jax.experimental.pallas import tpu as pltpu
from jax.experimental.pallas import tpu_sc as plsc

W_DIM = 64
BATCH = 4096
SEQ = 200
NUM_IDX = BATCH * SEQ
NC, NS = 2, 16
NW = NC * NS
CHUNK = 128
ROWS = NUM_IDX // CHUNK
RPW = ROWS // NW
K = 5
G = RPW // K


def _gather_body(idx_hbm, table_hbm, out_hbm, idx_v, rows0, rows1, sem_g0,
                 sem_g1, sem_o0, sem_o1):
    wid = lax.axis_index("s") * NC + lax.axis_index("c")
    base_row = wid * RPW
    rows_bufs = (rows0, rows1)
    gather_sems = (sem_g0, sem_g1)
    out_sems = (sem_o0, sem_o1)

    pltpu.sync_copy(idx_hbm.at[pl.ds(base_row, RPW)], idx_v)

    def fire_gathers(t, b):
        for j in range(K):
            pltpu.async_copy(
                table_hbm.at[idx_v.at[t * K + j]],
                rows_bufs[b].at[pl.ds(j * CHUNK, CHUNK)],
                gather_sems[b],
            )

    def wait_gathers(b):
        for j in range(K):
            pltpu.make_async_copy(
                table_hbm.at[idx_v.at[0]],
                rows_bufs[b].at[pl.ds(j * CHUNK, CHUNK)],
                gather_sems[b],
            ).wait()

    def fire_out(t, b):
        r0 = base_row + t * K
        pltpu.async_copy(
            rows_bufs[b],
            out_hbm.at[pl.ds(r0 * CHUNK, K * CHUNK)],
            out_sems[b],
        )

    def wait_out(t, b):
        r0 = base_row + t * K
        pltpu.make_async_copy(
            rows_bufs[b],
            out_hbm.at[pl.ds(r0 * CHUNK, K * CHUNK)],
            out_sems[b],
        ).wait()

    def substep(t, b):
        @pl.when(t >= 2)
        def _():
            wait_out(t - 2, b)

        fire_gathers(t, b)

        @pl.when(t >= 1)
        def _():
            wait_gathers(1 - b)
            fire_out(t - 1, 1 - b)

    def step(tp, carry):
        substep(2 * tp, 0)
        substep(2 * tp + 1, 1)
        return carry

    lax.fori_loop(0, G // 2, step, 0)

    wait_gathers(1)
    fire_out(G - 1, 1)
    wait_out(G - 2, 0)
    wait_out(G - 1, 1)


_sc_gather = functools.partial(
    pl.kernel,
    mesh=plsc.VectorSubcoreMesh(core_axis_name="c", subcore_axis_name="s"),
    out_type=jax.ShapeDtypeStruct((NUM_IDX, W_DIM), jnp.float32),
    scratch_types=[
        pltpu.VMEM((RPW, CHUNK), jnp.int32),
        pltpu.VMEM((K * CHUNK, W_DIM), jnp.float32),
        pltpu.VMEM((K * CHUNK, W_DIM), jnp.float32),
        pltpu.SemaphoreType.DMA,
        pltpu.SemaphoreType.DMA,
        pltpu.SemaphoreType.DMA,
        pltpu.SemaphoreType.DMA,
    ],
    compiler_params=pltpu.CompilerParams(use_tc_tiling_on_sc=False),
)(_gather_body)


def _mask_body(idx_ref, mask_ref):
    mask_ref[...] = idx_ref[...] != 0


def _make_mask(idx2d):
    return pl.pallas_call(
        _mask_body,
        out_shape=jax.ShapeDtypeStruct(idx2d.shape, jnp.bool_),
    )(idx2d)


def kernel(input_var, w_embedding):
    idx2d = input_var.reshape(ROWS, CHUNK)
    embedded = _sc_gather(idx2d, w_embedding)
    mask = _make_mask(idx2d)
    embedded = embedded.reshape(BATCH, SEQ, W_DIM)
    src_mask = mask.reshape(BATCH, SEQ)[:, None, None, :]
    return (embedded, src_mask)
